# baseline (speedup 1.0000x reference)
import functools

import jax
import jax.numpy as jnp
from jax import lax
from jax.experimental import pallas as pl
from jax.experimental.pallas import tpu as pltpu
from jax.experimental.pallas import tpu_sc as plsc

_VOCAB = 100000
_DIM = 128
_B, _S = 4, 4096
_NUM_LAYERS = 12

_NTOT = _B * _S
_NC, _NS = 2, 16
_NW = _NC * _NS
_PER_W = _NTOT // _NW
_CHUNK = 128
_ROWS_W = _PER_W // _CHUNK
_NBUF = 4
_NGIF = 2

_BLK = 1024


def _gather_one(idx2d, table):
    mesh = plsc.VectorSubcoreMesh(core_axis_name="c", subcore_axis_name="s")

    @functools.partial(
        pl.kernel,
        mesh=mesh,
        out_type=jax.ShapeDtypeStruct((_NTOT, _DIM), jnp.float32),
        scratch_types=[
            pltpu.VMEM((_ROWS_W, _CHUNK), jnp.int32),
            pltpu.VMEM((_NBUF, _CHUNK, _DIM), jnp.float32),
            pltpu.SemaphoreType.DMA((_NBUF,)),
            pltpu.SemaphoreType.DMA((_NBUF,)),
        ],
    )
    def body(idx_hbm, tab, out, idx_v, bufs, gsem, wsem):
        wid = lax.axis_index("s") * _NC + lax.axis_index("c")
        pltpu.sync_copy(idx_hbm.at[pl.ds(wid * _ROWS_W, _ROWS_W)], idx_v)

        n = _ROWS_W

        def issue_gather(j):
            return pltpu.async_copy(
                tab.at[idx_v.at[j]], bufs.at[j % _NBUF], gsem.at[j % _NBUF])

        gh = [None] * n
        wh = [None] * n
        for j in range(_NGIF):
            gh[j] = issue_gather(j)
        for j in range(n):
            gh[j].wait()
            row0 = wid * _PER_W + j * _CHUNK
            wh[j] = pltpu.async_copy(
                bufs.at[j % _NBUF], out.at[pl.ds(row0, _CHUNK)],
                wsem.at[j % _NBUF])
            nxt = j + _NGIF
            if nxt < n:
                if nxt >= _NBUF:
                    wh[nxt - _NBUF].wait()
                gh[nxt] = issue_gather(nxt)
        for j in range(max(0, n - _NBUF), n):
            wh[j].wait()

    return body(idx2d, table)


def _dup3(src):
    def body(x_hbm, a_hbm, b_hbm, c_hbm, sem):
        hs = [pltpu.make_async_copy(x_hbm, o, sem.at[i])
              for i, o in enumerate((a_hbm, b_hbm, c_hbm))]
        for h in hs:
            h.start()
        for h in hs:
            h.wait()

    return pl.pallas_call(
        body,
        in_specs=[pl.BlockSpec(memory_space=pltpu.MemorySpace.HBM)],
        out_specs=[pl.BlockSpec(memory_space=pltpu.MemorySpace.HBM)] * 3,
        out_shape=[jax.ShapeDtypeStruct((_NTOT, _DIM), jnp.float32)] * 3,
        scratch_shapes=[pltpu.SemaphoreType.DMA((3,))],
    )(src)


def kernel(input_seq, W0, W1, W2):
    idx2d = input_seq.reshape(_NTOT // _CHUNK, _CHUNK)
    uniq = [_gather_one(idx2d, w) for w in (W0, W1, W2)]
    dups = [_dup3(o) for o in uniq]
    outs = list(uniq)
    for k in range(3):
        for t in range(3):
            outs.append(dups[t][k])
    return tuple(o.reshape(_B, _S, _DIM) for o in outs)


# speedup vs baseline: 29.9771x; 29.9771x over previous
import functools

import jax
import jax.numpy as jnp
from jax import lax
from jax.experimental import pallas as pl
from jax.experimental.pallas import tpu as pltpu
from jax.experimental.pallas import tpu_sc as plsc

_VOCAB = 100000
_DIM = 128
_B, _S = 4, 4096
_NUM_LAYERS = 12

_NTOT = _B * _S
_NC, _NS = 2, 16
_NW = _NC * _NS
_PER_W = _NTOT // _NW
_CHUNK = 128
_ROWS_W = _PER_W // _CHUNK
_NBUF = 4
_NGIF = 2

_BLK = 2048


def _gather_one(idx2d, table):
    mesh = plsc.VectorSubcoreMesh(core_axis_name="c", subcore_axis_name="s")

    @functools.partial(
        pl.kernel,
        mesh=mesh,
        out_type=jax.ShapeDtypeStruct((_NTOT, _DIM), jnp.float32),
        scratch_types=[
            pltpu.VMEM((_ROWS_W, _CHUNK), jnp.int32),
            pltpu.VMEM((_NBUF, _CHUNK, _DIM), jnp.float32),
            pltpu.SemaphoreType.DMA((_NBUF,)),
            pltpu.SemaphoreType.DMA((_NBUF,)),
        ],
    )
    def body(idx_hbm, tab, out, idx_v, bufs, gsem, wsem):
        wid = lax.axis_index("s") * _NC + lax.axis_index("c")
        pltpu.sync_copy(idx_hbm.at[pl.ds(wid * _ROWS_W, _ROWS_W)], idx_v)

        n = _ROWS_W

        def issue_gather(j):
            return pltpu.async_copy(
                tab.at[idx_v.at[j]], bufs.at[j % _NBUF], gsem.at[j % _NBUF])

        gh = [None] * n
        wh = [None] * n
        for j in range(_NGIF):
            gh[j] = issue_gather(j)
        for j in range(n):
            gh[j].wait()
            row0 = wid * _PER_W + j * _CHUNK
            wh[j] = pltpu.async_copy(
                bufs.at[j % _NBUF], out.at[pl.ds(row0, _CHUNK)],
                wsem.at[j % _NBUF])
            nxt = j + _NGIF
            if nxt < n:
                if nxt >= _NBUF:
                    wh[nxt - _NBUF].wait()
                gh[nxt] = issue_gather(nxt)
        for j in range(max(0, n - _NBUF), n):
            wh[j].wait()

    return body(idx2d, table)


def _dup3(src):
    def body(x_ref, a_ref, b_ref, c_ref):
        v = x_ref[...]
        a_ref[...] = v
        b_ref[...] = v
        c_ref[...] = v

    spec = pl.BlockSpec((_BLK, _DIM), lambda i: (i, 0))
    return pl.pallas_call(
        body,
        grid=(_NTOT // _BLK,),
        in_specs=[spec],
        out_specs=[spec] * 3,
        out_shape=[jax.ShapeDtypeStruct((_NTOT, _DIM), jnp.float32)] * 3,
    )(src)


def kernel(input_seq, W0, W1, W2):
    idx2d = input_seq.reshape(_NTOT // _CHUNK, _CHUNK)
    uniq = [_gather_one(idx2d, w) for w in (W0, W1, W2)]
    dups = [_dup3(o) for o in uniq]
    outs = list(uniq)
    for k in range(3):
        for t in range(3):
            outs.append(dups[t][k])
    return tuple(o.reshape(_B, _S, _DIM) for o in outs)


# speedup vs baseline: 36.0215x; 1.2016x over previous
import functools

import jax
import jax.numpy as jnp
from jax import lax
from jax.experimental import pallas as pl
from jax.experimental.pallas import tpu as pltpu
from jax.experimental.pallas import tpu_sc as plsc

_VOCAB = 100000
_DIM = 128
_B, _S = 4, 4096
_NUM_LAYERS = 12

_NTOT = _B * _S
_NC, _NS = 2, 16
_NW = _NC * _NS
_PER_W = _NTOT // _NW
_CHUNK = 128
_ROWS_W = _PER_W // _CHUNK
_WBUF = 256
_NBUF = 3
_NGIF = 2


def _gather3(idx2d, w0, w1, w2):
    mesh = plsc.VectorSubcoreMesh(core_axis_name="c", subcore_axis_name="s")

    @functools.partial(
        pl.kernel,
        mesh=mesh,
        out_type=[jax.ShapeDtypeStruct((_NTOT, _DIM), jnp.float32)] * _NUM_LAYERS,
        scratch_types=[
            pltpu.VMEM((_ROWS_W, _CHUNK), jnp.int32),
            pltpu.VMEM((_NBUF, _WBUF, _DIM), jnp.float32),
            pltpu.SemaphoreType.DMA((_NBUF,)),
            pltpu.SemaphoreType.DMA((_NBUF,)),
        ],
    )
    def body(idx_hbm, t0, t1, t2, *rest):
        outs = rest[:_NUM_LAYERS]
        idx_v, bufs, gsem, wsem = rest[_NUM_LAYERS:]
        wid = lax.axis_index("s") * _NC + lax.axis_index("c")
        pltpu.sync_copy(idx_hbm.at[pl.ds(wid * _ROWS_W, _ROWS_W)], idx_v)

        tabs = (t0, t1, t2)
        cpb = _WBUF // _CHUNK
        tasks = [(t, b) for t in range(3) for b in range(_ROWS_W // cpb)]
        n = len(tasks)

        def issue_gathers(k):
            t, b = tasks[k]
            slot = k % _NBUF
            return [
                pltpu.async_copy(
                    tabs[t].at[idx_v.at[b * cpb + h]],
                    bufs.at[slot].at[pl.ds(h * _CHUNK, _CHUNK)],
                    gsem.at[slot])
                for h in range(cpb)
            ]

        def issue_writebacks(k):
            t, b = tasks[k]
            slot = k % _NBUF
            row0 = wid * _PER_W + b * _WBUF
            return [
                pltpu.async_copy(
                    bufs.at[slot], outs[l].at[pl.ds(row0, _WBUF)],
                    wsem.at[slot])
                for l in range(t, _NUM_LAYERS, 3)
            ]

        gh = [None] * n
        wh = [None] * n
        for k in range(_NGIF):
            gh[k] = issue_gathers(k)
        for k in range(n):
            for h in gh[k]:
                h.wait()
            wh[k] = issue_writebacks(k)
            nxt = k + _NGIF
            if nxt < n:
                if nxt >= _NBUF:
                    for h in wh[nxt - _NBUF]:
                        h.wait()
                gh[nxt] = issue_gathers(nxt)
        for k in range(n - _NBUF, n):
            for h in wh[k]:
                h.wait()

    return body(idx2d, w0, w1, w2)


def kernel(input_seq, W0, W1, W2):
    idx2d = input_seq.reshape(_NTOT // _CHUNK, _CHUNK)
    outs = _gather3(idx2d, W0, W1, W2)
    return tuple(o.reshape(_B, _S, _DIM) for o in outs)


# speedup vs baseline: 36.0259x; 1.0001x over previous
import functools

import jax
import jax.numpy as jnp
from jax import lax
from jax.experimental import pallas as pl
from jax.experimental.pallas import tpu as pltpu
from jax.experimental.pallas import tpu_sc as plsc

_VOCAB = 100000
_DIM = 128
_B, _S = 4, 4096
_NUM_LAYERS = 12

_NTOT = _B * _S
_NC, _NS = 2, 16
_NW = _NC * _NS
_PER_W = _NTOT // _NW
_CHUNK = 128
_ROWS_W = _PER_W // _CHUNK
_WBUF = 256
_NBUF = 3
_NGIF = 2


def _gather3(idx2d, w0, w1, w2):
    mesh = plsc.VectorSubcoreMesh(core_axis_name="c", subcore_axis_name="s")

    @functools.partial(
        pl.kernel,
        mesh=mesh,
        out_type=[jax.ShapeDtypeStruct((_NTOT, _DIM), jnp.float32)] * _NUM_LAYERS,
        scratch_types=[
            pltpu.VMEM((_PER_W,), jnp.int32),
            pltpu.VMEM((_NBUF, _WBUF, _DIM), jnp.float32),
            pltpu.SemaphoreType.DMA((_NBUF,)),
            pltpu.SemaphoreType.DMA((_NBUF,)),
        ],
    )
    def body(idx_hbm, t0, t1, t2, *rest):
        outs = rest[:_NUM_LAYERS]
        idx_v, bufs, gsem, wsem = rest[_NUM_LAYERS:]
        wid = lax.axis_index("s") * _NC + lax.axis_index("c")
        pltpu.sync_copy(
            idx_hbm.at[wid // (_S // _PER_W), pl.ds((wid % (_S // _PER_W)) * _PER_W, _PER_W)],
            idx_v)

        tabs = (t0, t1, t2)
        cpb = _WBUF // _CHUNK
        tasks = [(t, b) for t in range(3) for b in range(_ROWS_W // cpb)]
        n = len(tasks)

        def issue_gathers(k):
            t, b = tasks[k]
            slot = k % _NBUF
            return [
                pltpu.async_copy(
                    tabs[t].at[idx_v.at[pl.ds((b * cpb + h) * _CHUNK, _CHUNK)]],
                    bufs.at[slot].at[pl.ds(h * _CHUNK, _CHUNK)],
                    gsem.at[slot])
                for h in range(cpb)
            ]

        def issue_writebacks(k):
            t, b = tasks[k]
            slot = k % _NBUF
            row0 = wid * _PER_W + b * _WBUF
            return [
                pltpu.async_copy(
                    bufs.at[slot], outs[l].at[pl.ds(row0, _WBUF)],
                    wsem.at[slot])
                for l in range(t, _NUM_LAYERS, 3)
            ]

        gh = [None] * n
        wh = [None] * n
        for k in range(_NGIF):
            gh[k] = issue_gathers(k)
        for k in range(n):
            for h in gh[k]:
                h.wait()
            wh[k] = issue_writebacks(k)
            nxt = k + _NGIF
            if nxt < n:
                if nxt >= _NBUF:
                    for h in wh[nxt - _NBUF]:
                        h.wait()
                gh[nxt] = issue_gathers(nxt)
        for k in range(n - _NBUF, n):
            for h in wh[k]:
                h.wait()

    return body(idx2d, w0, w1, w2)


def kernel(input_seq, W0, W1, W2):
    outs = _gather3(input_seq, W0, W1, W2)
    return tuple(o.reshape(_B, _S, _DIM) for o in outs)
